# baseline (speedup 1.0000x reference)
import functools

import jax
import jax.numpy as jnp
from jax import lax
from jax.experimental import pallas as pl
from jax.experimental.pallas import tpu as pltpu
from jax.experimental.pallas import tpu_sc as plsc

_VOCAB = 1000000
_D = 16
_B = 16384
_EPS = 1e-8
_BLK = 8000
_NBLK = _VOCAB // _BLK


@functools.cache
def _make_sc_gather():
    info = plsc.get_sparse_core_info()
    nc, ns = info.num_cores, info.num_subcores
    nw = nc * ns
    bpw = _B // nw
    mesh = plsc.VectorSubcoreMesh(core_axis_name="c", subcore_axis_name="s")
    rows = jax.ShapeDtypeStruct((_B, _D), jnp.float32)

    @functools.partial(
        pl.kernel,
        mesh=mesh,
        out_type=(rows, rows, rows, rows),
        scratch_types=[
            pltpu.VMEM((bpw,), jnp.int32),
            pltpu.VMEM((bpw,), jnp.int32),
            pltpu.VMEM((bpw, _D), jnp.float32),
            pltpu.VMEM((bpw, _D), jnp.float32),
            pltpu.VMEM((bpw, _D), jnp.float32),
            pltpu.VMEM((bpw, _D), jnp.float32),
            pltpu.SemaphoreType.DMA,
        ],
        compiler_params=pltpu.CompilerParams(use_tc_tiling_on_sc=False),
    )
    def gather(min_hbm, del_hbm, i1_hbm, i2_hbm,
               o1m, o1d, o2m, o2d,
               i1_v, i2_v, b1, b2, b3, b4, sem):
        wid = lax.axis_index("s") * nc + lax.axis_index("c")
        base = wid * bpw
        pltpu.sync_copy(i1_hbm.at[pl.ds(base, bpw)], i1_v)
        pltpu.sync_copy(i2_hbm.at[pl.ds(base, bpw)], i2_v)
        c1 = pltpu.async_copy(min_hbm.at[i1_v], b1, sem)
        c2 = pltpu.async_copy(del_hbm.at[i1_v], b2, sem)
        c3 = pltpu.async_copy(min_hbm.at[i2_v], b3, sem)
        c4 = pltpu.async_copy(del_hbm.at[i2_v], b4, sem)
        c1.wait()
        c2.wait()
        c3.wait()
        c4.wait()
        pltpu.sync_copy(b1, o1m.at[pl.ds(base, bpw)])
        pltpu.sync_copy(b2, o1d.at[pl.ds(base, bpw)])
        pltpu.sync_copy(b3, o2m.at[pl.ds(base, bpw)])
        pltpu.sync_copy(b4, o2d.at[pl.ds(base, bpw)])

    return gather


_BCH = 2048
_NBCH = _B // _BCH


def _loss_body(t1m, t1d, t2m, t2d, lab, minb, delb, out_ref):
    i = pl.program_id(0)
    reg = jnp.sum(jnp.abs(1.0 - minb[...] - delb[...]))

    @pl.when(i == 0)
    def _():
        out_ref[0, 0] = 0.0

    @pl.when(i < _NBCH)
    def _():
        a_lo = t1m[...]
        a_hi = a_lo + t1d[...]
        b_lo = t2m[...]
        b_hi = b_lo + t2d[...]
        meet_lo = jnp.maximum(a_lo, b_lo)
        meet_hi = jnp.minimum(a_hi, b_hi)
        join_lo = jnp.minimum(a_lo, b_lo)
        join_hi = jnp.maximum(a_hi, b_hi)
        disj = jnp.any(meet_hi <= meet_lo, axis=1, keepdims=True)

        def lv(lo, hi):
            return jnp.sum(jnp.log(jnp.clip(hi - lo, _EPS, None)),
                           axis=1, keepdims=True)

        log_meet = lv(meet_lo, meet_hi)
        log_join = lv(join_lo, join_hi)
        log_t1 = lv(a_lo, a_hi)
        log_t2 = lv(b_lo, b_hi)
        cond = log_meet - log_t2
        pos_overlap = -cond
        upper = jnp.clip(jnp.exp(log_join) - jnp.exp(log_t1) - jnp.exp(log_t2),
                         _EPS, None)
        pos_disjoint = -(jnp.log(upper) - log_t2)
        train_pos = jnp.where(disj, pos_disjoint, pos_overlap)
        neg_overlap = -jnp.log(jnp.clip(1.0 - jnp.exp(cond), _EPS, None))
        train_neg = jnp.where(disj, 0.0, neg_overlap)
        lb = lab[...]
        cond_loss = (jnp.sum(train_pos * lb)
                     + jnp.sum(train_neg * (1.0 - lb))) / (_B / 2)
        out_ref[0, 0] += cond_loss

    out_ref[0, 0] += reg * (0.0001 / _VOCAB)


def kernel(t1x, t2x, label, min_embed, delta_embed):
    i1 = t1x[:, 0].astype(jnp.int32)
    i2 = t2x[:, 0].astype(jnp.int32)
    t1m, t1d, t2m, t2d = _make_sc_gather()(min_embed, delta_embed, i1, i2)
    lab = label.reshape(_B, 1)

    batch_spec = pl.BlockSpec((_BCH, _D), lambda i: (jnp.minimum(i, _NBCH - 1), 0))
    lab_spec = pl.BlockSpec((_BCH, 1), lambda i: (jnp.minimum(i, _NBCH - 1), 0))
    tab_spec = pl.BlockSpec((_BLK, _D), lambda i: (i, 0))
    loss = pl.pallas_call(
        _loss_body,
        grid=(_NBLK,),
        in_specs=[batch_spec, batch_spec, batch_spec, batch_spec,
                  lab_spec, tab_spec, tab_spec],
        out_specs=pl.BlockSpec(memory_space=pltpu.SMEM),
        out_shape=jax.ShapeDtypeStruct((1, 1), jnp.float32),
    )(t1m, t1d, t2m, t2d, lab, min_embed, delta_embed)
    return loss[0, 0]


# speedup vs baseline: 1.0156x; 1.0156x over previous
import functools

import jax
import jax.numpy as jnp
from jax import lax
from jax.experimental import pallas as pl
from jax.experimental.pallas import tpu as pltpu
from jax.experimental.pallas import tpu_sc as plsc

_VOCAB = 1000000
_D = 16
_B = 16384
_EPS = 1e-8
_G = 128 // _D
_BR = _B // _G
_TROWS = _VOCAB // _G
_TBLK = 5000
_TN = _TROWS // _TBLK


@functools.cache
def _make_sc_gather():
    info = plsc.get_sparse_core_info()
    nc, ns = info.num_cores, info.num_subcores
    nw = nc * ns
    bpw = _B // nw
    mesh = plsc.VectorSubcoreMesh(core_axis_name="c", subcore_axis_name="s")
    rows = jax.ShapeDtypeStruct((_B, _D), jnp.float32)

    @functools.partial(
        pl.kernel,
        mesh=mesh,
        out_type=(rows, rows, rows, rows),
        scratch_types=[
            pltpu.VMEM((bpw,), jnp.int32),
            pltpu.VMEM((bpw,), jnp.int32),
            pltpu.VMEM((bpw, _D), jnp.float32),
            pltpu.VMEM((bpw, _D), jnp.float32),
            pltpu.VMEM((bpw, _D), jnp.float32),
            pltpu.VMEM((bpw, _D), jnp.float32),
            pltpu.SemaphoreType.DMA,
        ],
        compiler_params=pltpu.CompilerParams(use_tc_tiling_on_sc=False),
    )
    def gather(min_hbm, del_hbm, i1_hbm, i2_hbm,
               o1m, o1d, o2m, o2d,
               i1_v, i2_v, b1, b2, b3, b4, sem):
        wid = lax.axis_index("s") * nc + lax.axis_index("c")
        base = wid * bpw
        pltpu.sync_copy(i1_hbm.at[pl.ds(base, bpw)], i1_v)
        pltpu.sync_copy(i2_hbm.at[pl.ds(base, bpw)], i2_v)
        c1 = pltpu.async_copy(min_hbm.at[i1_v], b1, sem)
        c2 = pltpu.async_copy(del_hbm.at[i1_v], b2, sem)
        c3 = pltpu.async_copy(min_hbm.at[i2_v], b3, sem)
        c4 = pltpu.async_copy(del_hbm.at[i2_v], b4, sem)
        c1.wait()
        c2.wait()
        c3.wait()
        c4.wait()
        pltpu.sync_copy(b1, o1m.at[pl.ds(base, bpw)])
        pltpu.sync_copy(b2, o1d.at[pl.ds(base, bpw)])
        pltpu.sync_copy(b3, o2m.at[pl.ds(base, bpw)])
        pltpu.sync_copy(b4, o2d.at[pl.ds(base, bpw)])

    return gather


def _seg_sum(x, sel):
    return lax.dot_general(x, sel, (((1,), (0,)), ((), ())),
                           precision=lax.Precision.HIGHEST,
                           preferred_element_type=jnp.float32)


def _loss_body(t1m, t1d, t2m, t2d, lab, minb, delb, out_ref):
    i = pl.program_id(0)
    reg = jnp.sum(jnp.abs(1.0 - minb[...] - delb[...]))

    @pl.when(i == 0)
    def _():
        lane_grp = lax.broadcasted_iota(jnp.int32, (128, _G), 0) // _D
        grp = lax.broadcasted_iota(jnp.int32, (128, _G), 1)
        sel = (lane_grp == grp).astype(jnp.float32)

        a_lo = t1m[...]
        a_hi = a_lo + t1d[...]
        b_lo = t2m[...]
        b_hi = b_lo + t2d[...]
        meet_lo = jnp.maximum(a_lo, b_lo)
        meet_hi = jnp.minimum(a_hi, b_hi)
        join_lo = jnp.minimum(a_lo, b_lo)
        join_hi = jnp.maximum(a_hi, b_hi)

        def lv(lo, hi):
            return _seg_sum(jnp.log(jnp.clip(hi - lo, _EPS, None)), sel)

        log_meet = lv(meet_lo, meet_hi)
        log_join = lv(join_lo, join_hi)
        log_t1 = lv(a_lo, a_hi)
        log_t2 = lv(b_lo, b_hi)
        disj_cnt = _seg_sum((meet_hi <= meet_lo).astype(jnp.float32), sel)
        disj = disj_cnt > 0.0

        cond = log_meet - log_t2
        pos_overlap = -cond
        upper = jnp.clip(jnp.exp(log_join) - jnp.exp(log_t1) - jnp.exp(log_t2),
                         _EPS, None)
        pos_disjoint = -(jnp.log(upper) - log_t2)
        train_pos = jnp.where(disj, pos_disjoint, pos_overlap)
        neg_overlap = -jnp.log(jnp.clip(1.0 - jnp.exp(cond), _EPS, None))
        train_neg = jnp.where(disj, 0.0, neg_overlap)
        lb = lab[...]
        cond_loss = (jnp.sum(train_pos * lb)
                     + jnp.sum(train_neg * (1.0 - lb))) / (_B / 2)
        out_ref[0, 0] = cond_loss

    out_ref[0, 0] += reg * (0.0001 / _VOCAB)


def kernel(t1x, t2x, label, min_embed, delta_embed):
    i1 = t1x[:, 0].astype(jnp.int32)
    i2 = t2x[:, 0].astype(jnp.int32)
    t1m, t1d, t2m, t2d = _make_sc_gather()(min_embed, delta_embed, i1, i2)
    lab = label.reshape(_BR, _G)
    wide = lambda a: a.reshape(_BR, 128)
    min_w = min_embed.reshape(_TROWS, 128)
    del_w = delta_embed.reshape(_TROWS, 128)

    batch_spec = pl.BlockSpec((_BR, 128), lambda i: (0, 0))
    lab_spec = pl.BlockSpec((_BR, _G), lambda i: (0, 0))
    tab_spec = pl.BlockSpec((_TBLK, 128), lambda i: (i, 0))
    loss = pl.pallas_call(
        _loss_body,
        grid=(_TN,),
        in_specs=[batch_spec, batch_spec, batch_spec, batch_spec,
                  lab_spec, tab_spec, tab_spec],
        out_specs=pl.BlockSpec(memory_space=pltpu.SMEM),
        out_shape=jax.ShapeDtypeStruct((1, 1), jnp.float32),
    )(wide(t1m), wide(t1d), wide(t2m), wide(t2d), lab, min_w, del_w)
    return loss[0, 0]
